# baseline (speedup 1.0000x reference)
import functools

import jax
import jax.numpy as jnp
from jax import lax
from jax.experimental import pallas as pl
from jax.experimental.pallas import tpu as pltpu
from jax.experimental.pallas import tpu_sc as plsc

L_Q = 2048
L_V = 2048
N_EMB = 129
D = 64
MAXP = (N_EMB - 1) // 2
FILL_LO = L_V - 1 - MAXP
B_ROWS = 4096

NC = 2
NS = 16
NW = NC * NS
CHUNK = B_ROWS // NS
ROWS_PER_W = L_Q // NW


def _sc_band_kernel(emb_hbm, out_hbm, table_v, stage_v, b_sh, dma_sem):
    c = lax.axis_index("c")
    s = lax.axis_index("s")

    pltpu.sync_copy(emb_hbm, table_v)

    base = s * CHUNK

    def build_row(r, _):
        t = jnp.clip(base + r - FILL_LO, 0, N_EMB - 1)
        for col in range(D // 16):
            sl = pl.ds(col * 16, 16)
            stage_v[r, sl] = table_v[t, sl]
        return _

    lax.fori_loop(0, CHUNK, build_row, 0)
    pltpu.sync_copy(stage_v, b_sh.at[pl.ds(base, CHUNK)])

    plsc.subcore_barrier()

    wid = s * NC + c
    i0 = wid * ROWS_PER_W

    handles = []
    for r in range(ROWS_PER_W):
        i = i0 + r
        h = pltpu.make_async_copy(
            b_sh.at[pl.ds(L_V - 1 - i, L_V)], out_hbm.at[i], dma_sem
        )
        h.start()
        handles.append(h)
    for h in handles:
        h.wait()

    fixes = []
    for r in range(ROWS_PER_W):
        i = i0 + r
        j0a = jnp.minimum((i + 1) // 8 * 8, L_V - 16)
        h = pltpu.make_async_copy(
            table_v.at[pl.ds(j0a - i + MAXP, 16)],
            out_hbm.at[i, pl.ds(j0a, 16)],
            dma_sem,
        )
        h.start()
        fixes.append(h)
    for h in fixes:
        h.wait()


def kernel(query, value, embeddings):
    del query, value
    mesh = plsc.VectorSubcoreMesh(core_axis_name="c", subcore_axis_name="s")
    f = functools.partial(
        pl.kernel,
        mesh=mesh,
        out_type=jax.ShapeDtypeStruct((L_Q, L_V, D), jnp.float32),
        scratch_types=[
            pltpu.VMEM((N_EMB, D), jnp.float32),
            pltpu.VMEM((CHUNK, D), jnp.float32),
            pltpu.VMEM_SHARED((B_ROWS, D), jnp.float32),
            pltpu.SemaphoreType.DMA,
        ],
    )(_sc_band_kernel)
    return f(embeddings)


# speedup vs baseline: 1.2100x; 1.2100x over previous
import functools

import jax
import jax.numpy as jnp
from jax import lax
from jax.experimental import pallas as pl
from jax.experimental.pallas import tpu as pltpu
from jax.experimental.pallas import tpu_sc as plsc

L_Q = 2048
L_V = 2048
N_EMB = 129
D = 64
MAXP = (N_EMB - 1) // 2
FILL_LO = L_V - 1 - MAXP
B_ROWS = 4096
NPH = 8
PH_ROWS = 4088

NC = 2
NS = 16
CHUNK = B_ROWS // NS

RPS = 8
TC_STEPS = L_Q // RPS


def _sc_phase_body(emb_hbm, b8_hbm, table_v, stage_v, b_sh):
    c = lax.axis_index("c")
    s = lax.axis_index("s")

    pltpu.sync_copy(emb_hbm, table_v)

    base = s * CHUNK

    def build_row(r, _):
        t = jnp.clip(base + r - FILL_LO, 0, N_EMB - 1)
        for col in range(D // 16):
            sl = pl.ds(col * 16, 16)
            stage_v[r, sl] = table_v[t, sl]
        return _

    lax.fori_loop(0, CHUNK, build_row, 0)
    pltpu.sync_copy(stage_v, b_sh.at[pl.ds(base, CHUNK)])
    plsc.subcore_barrier()

    for p in range(NPH):
        own = jnp.logical_and(c == p // 4, s == p % 4)

        @pl.when(own)
        def _(p=p):
            pltpu.sync_copy(b_sh.at[pl.ds(p, PH_ROWS)], b8_hbm.at[p])
            lo = 2048 - p
            ta = (lo // 8) * 8
            pltpu.sync_copy(
                table_v.at[pl.ds(ta + p - FILL_LO, 16)],
                b8_hbm.at[p, pl.ds(ta, 16)],
            )


def _tc_fanout_body(b8_ref, out_ref, sem):
    q = pl.program_id(0)

    def row_dmas(qq):
        hs = []
        for rr in range(RPS):
            i = qq * RPS + rr
            p = (L_V - 1 - rr) % NPH
            s8 = L_V - 1 - i - p
            hs.append(
                pltpu.make_async_copy(
                    b8_ref.at[p, pl.ds(s8, L_V), :], out_ref.at[i], sem
                )
            )
        return hs

    for h in row_dmas(q):
        h.start()

    @pl.when(q > 0)
    def _():
        for h in row_dmas(q - 1):
            h.wait()

    @pl.when(q == TC_STEPS - 1)
    def _():
        for h in row_dmas(q):
            h.wait()


def kernel(query, value, embeddings):
    del query, value
    mesh = plsc.VectorSubcoreMesh(core_axis_name="c", subcore_axis_name="s")
    sc_phase = functools.partial(
        pl.kernel,
        mesh=mesh,
        out_type=jax.ShapeDtypeStruct((NPH, PH_ROWS, D), jnp.float32),
        scratch_types=[
            pltpu.VMEM((N_EMB, D), jnp.float32),
            pltpu.VMEM((CHUNK, D), jnp.float32),
            pltpu.VMEM_SHARED((B_ROWS, D), jnp.float32),
        ],
    )(_sc_phase_body)
    b8 = sc_phase(embeddings)

    fanout = pl.pallas_call(
        _tc_fanout_body,
        grid=(TC_STEPS,),
        in_specs=[pl.BlockSpec(memory_space=pltpu.VMEM)],
        out_specs=pl.BlockSpec(memory_space=pltpu.HBM),
        out_shape=jax.ShapeDtypeStruct((L_Q, L_V, D), jnp.float32),
        scratch_shapes=[pltpu.SemaphoreType.DMA],
    )
    return fanout(b8)
